# baseline (speedup 1.0000x reference)
import functools

import jax
import jax.numpy as jnp
from jax.experimental import pallas as pl
from jax.experimental.pallas import tpu as pltpu

_ALPHA = 0.25
_B = 4096
_PACK = 8


def _loss_kernel(num_blocks, num_classes,
                 x_ref, ctm_ref, ctr_ref, pb_ref, bt_ref, pi_ref, ti_ref,
                 cls_ref, reg_ref, iou_ref, acc_ref):
    g = pl.program_id(0)
    w = _PACK * num_classes

    ct8 = ctm_ref[:, 0:_PACK]
    mk8 = ctm_ref[:, _PACK:2 * _PACK]
    valid8 = jnp.where(ct8 >= 0.0, 1.0, 0.0) * (1.0 - mk8)

    groups = jnp.arange(w, dtype=jnp.int32) // num_classes
    expand = (groups[None, :] == jnp.arange(_PACK, dtype=jnp.int32)[:, None])
    expand = expand.astype(jnp.bfloat16)
    dot = functools.partial(
        jax.lax.dot_general,
        dimension_numbers=(((1,), (0,)), ((), ())),
        preferred_element_type=jnp.float32)
    ctexp = dot(ct8.astype(jnp.bfloat16), expand)
    vexp = dot(valid8.astype(jnp.bfloat16), expand)

    x = x_ref[...]
    classpat = (jnp.arange(w, dtype=jnp.int32) % num_classes
                ).astype(jnp.float32)[None, :]
    tau = ctexp == classpat
    a = jnp.exp(-jnp.abs(x))
    l1p = jnp.log1p(a)
    ce0 = jnp.maximum(x, 0.0) + l1p
    tx = jnp.where(tau, x, 0.0)
    ce = ce0 - tx
    arg = (ce0 - x) + tx
    fsq = jnp.exp(-2.0 * arg)
    at = jnp.where(tau, _ALPHA, 1.0 - _ALPHA)
    cls_part = jnp.sum(at * ce * fsq * vexp)

    ctr = ctr_ref[0]
    fg = jnp.where((ctr >= 0.0) & (ctr != float(num_classes)), 1.0, 0.0)
    fg_part = jnp.sum(fg)

    pb = pb_ref[...]
    bt = bt_ref[...]
    px0, py0, px1, py1 = pb[0:1], pb[1:2], pb[2:3], pb[3:4]
    tx0, ty0, tx1, ty1 = bt[0:1], bt[1:2], bt[2:3], bt[3:4]
    area1 = (px1 - px0) * (py1 - py0)
    area2 = (tx1 - tx0) * (ty1 - ty0)
    iw = jnp.clip(jnp.minimum(px1, tx1) - jnp.maximum(px0, tx0), 0.0, None)
    ih = jnp.clip(jnp.minimum(py1, ty1) - jnp.maximum(py0, ty0), 0.0, None)
    inter = iw * ih
    union = area1 + area2 - inter
    ew = jnp.maximum(px1, tx1) - jnp.minimum(px0, tx0)
    eh = jnp.maximum(py1, ty1) - jnp.minimum(py0, ty0)
    area_e = jnp.clip(ew, 0.0, None) * jnp.clip(eh, 0.0, None)
    giou = inter / union - (area_e - union) / area_e
    reg_part = jnp.sum((1.0 - giou) * fg)

    pi = pi_ref[0]
    ti = ti_ref[0]
    bce = jnp.maximum(pi, 0.0) - pi * ti + jnp.log1p(jnp.exp(-jnp.abs(pi)))
    iou_part = jnp.sum(bce * fg)

    part = jnp.concatenate(
        [jnp.full((1, 1), v, jnp.float32)
         for v in (cls_part, reg_part, iou_part, fg_part)], axis=1)

    @pl.when(g == 0)
    def _():
        acc_ref[...] = jnp.zeros((1, 4), jnp.float32)

    tot = acc_ref[...] + part
    acc_ref[...] = tot

    @pl.when(g == num_blocks - 1)
    def _():
        nf = jnp.maximum(tot[0:1, 3:4], 1.0)
        cls_ref[...] = tot[0:1, 0:1] / nf
        reg_ref[...] = tot[0:1, 1:2] / nf
        iou_ref[...] = tot[0:1, 2:3] / nf


def kernel(pred_cls, pred_box, pred_iou, cls_targets, box_targets,
           iou_targets, mask):
    n, c = pred_cls.shape
    b = _B
    nb = n // b
    rows = n // _PACK
    br = b // _PACK

    x640 = pred_cls.reshape(rows, _PACK * c)
    ct_f = cls_targets.astype(jnp.float32)
    ctm8 = jnp.concatenate([ct_f.reshape(rows, _PACK),
                            mask.astype(jnp.float32).reshape(rows, _PACK)],
                           axis=1)
    ct_row = ct_f.reshape(nb, 1, b)
    pb_t = pred_box.T
    bt_t = box_targets.T
    pi_row = pred_iou.reshape(nb, 1, b)
    ti_row = iou_targets.reshape(nb, 1, b)

    out = pl.pallas_call(
        functools.partial(_loss_kernel, nb, c),
        grid=(nb,),
        in_specs=[
            pl.BlockSpec((br, _PACK * c), lambda g: (g, 0)),
            pl.BlockSpec((br, 2 * _PACK), lambda g: (g, 0)),
            pl.BlockSpec((1, 1, b), lambda g: (g, 0, 0)),
            pl.BlockSpec((4, b), lambda g: (0, g)),
            pl.BlockSpec((4, b), lambda g: (0, g)),
            pl.BlockSpec((1, 1, b), lambda g: (g, 0, 0)),
            pl.BlockSpec((1, 1, b), lambda g: (g, 0, 0)),
        ],
        out_specs=[
            pl.BlockSpec((1, 1), lambda g: (0, 0)),
            pl.BlockSpec((1, 1), lambda g: (0, 0)),
            pl.BlockSpec((1, 1), lambda g: (0, 0)),
        ],
        out_shape=[jax.ShapeDtypeStruct((1, 1), jnp.float32)] * 3,
        scratch_shapes=[pltpu.VMEM((1, 4), jnp.float32)],
    )(x640, ctm8, ct_row, pb_t, bt_t, pi_row, ti_row)
    cls_s, reg_s, iou_s = out
    return (cls_s[0, 0], reg_s[0, 0], iou_s[0, 0])


# speedup vs baseline: 1.3402x; 1.3402x over previous
import functools

import jax
import jax.numpy as jnp
from jax.experimental import pallas as pl
from jax.experimental.pallas import tpu as pltpu

_ALPHA = 0.25
_B = 8192
_LOG2E = 1.4426950408889634
_LN2 = 0.6931471805599453


def _loss_kernel(num_blocks, num_classes,
                 x_ref, ctm_ref, pb_ref, bt_ref, pi_ref, ti_ref,
                 cls_ref, reg_ref, iou_ref, acc_ref):
    g = pl.program_id(0)

    ct = ctm_ref[0, 0:1, :]
    mval = ctm_ref[0, 1:2, :]
    fg = jnp.where((ct >= 0.0) & (ct != float(num_classes)), 1.0, 0.0)
    valid = jnp.where(ct >= 0.0, 1.0, 0.0) * (1.0 - mval)
    fg_part = jnp.sum(fg)

    ctv_col = jnp.concatenate([ct, valid], axis=0).T
    ct_col = ctv_col[:, 0:1]
    valid_col = ctv_col[:, 1:2]

    x = x_ref[...]
    lane = jax.lax.broadcasted_iota(jnp.int32, x.shape, 1).astype(jnp.float32)
    tau = (lane == ct_col).astype(jnp.float32)
    a = jnp.exp2(jnp.abs(x) * (-_LOG2E))
    l1p = jnp.log2(1.0 + a) * _LN2
    ce0 = jnp.maximum(x, 0.0) + l1p
    tx = tau * x
    ce = ce0 - tx
    arg = (ce0 - x) + tx
    fsq = jnp.exp2(arg * (-2.0 * _LOG2E))
    at = (1.0 - _ALPHA) - (1.0 - 2.0 * _ALPHA) * tau
    cls_part = jnp.sum(at * ce * fsq * valid_col)

    pb = pb_ref[...]
    bt = bt_ref[...]
    px0, py0, px1, py1 = pb[0:1], pb[1:2], pb[2:3], pb[3:4]
    tx0, ty0, tx1, ty1 = bt[0:1], bt[1:2], bt[2:3], bt[3:4]
    area1 = (px1 - px0) * (py1 - py0)
    area2 = (tx1 - tx0) * (ty1 - ty0)
    iw = jnp.maximum(jnp.minimum(px1, tx1) - jnp.maximum(px0, tx0), 0.0)
    ih = jnp.maximum(jnp.minimum(py1, ty1) - jnp.maximum(py0, ty0), 0.0)
    inter = iw * ih
    union = area1 + area2 - inter
    ew = jnp.maximum(px1, tx1) - jnp.minimum(px0, tx0)
    eh = jnp.maximum(py1, ty1) - jnp.minimum(py0, ty0)
    area_e = jnp.maximum(ew, 0.0) * jnp.maximum(eh, 0.0)
    giou = inter / union - (area_e - union) / area_e
    reg_part = jnp.sum((1.0 - giou) * fg)

    pi = pi_ref[0]
    ti = ti_ref[0]
    ai = jnp.exp2(jnp.abs(pi) * (-_LOG2E))
    bce = jnp.maximum(pi, 0.0) - pi * ti + jnp.log2(1.0 + ai) * _LN2
    iou_part = jnp.sum(bce * fg)

    part = jnp.concatenate(
        [jnp.full((1, 1), v, jnp.float32)
         for v in (cls_part, reg_part, iou_part, fg_part)], axis=1)

    @pl.when(g == 0)
    def _():
        acc_ref[...] = jnp.zeros((1, 4), jnp.float32)

    tot = acc_ref[...] + part
    acc_ref[...] = tot

    @pl.when(g == num_blocks - 1)
    def _():
        nf = jnp.maximum(tot[0:1, 3:4], 1.0)
        cls_ref[...] = tot[0:1, 0:1] / nf
        reg_ref[...] = tot[0:1, 1:2] / nf
        iou_ref[...] = tot[0:1, 2:3] / nf


def kernel(pred_cls, pred_box, pred_iou, cls_targets, box_targets,
           iou_targets, mask):
    n, c = pred_cls.shape
    b = _B
    nb = n // b
    ctm = jnp.stack([cls_targets.astype(jnp.float32),
                     mask.astype(jnp.float32)], axis=0)
    ctm = ctm.reshape(2, nb, b).transpose(1, 0, 2)
    pb_t = pred_box.T
    bt_t = box_targets.T
    pi_row = pred_iou.reshape(nb, 1, b)
    ti_row = iou_targets.reshape(nb, 1, b)

    out = pl.pallas_call(
        functools.partial(_loss_kernel, nb, c),
        grid=(nb,),
        in_specs=[
            pl.BlockSpec((b, c), lambda g: (g, 0)),
            pl.BlockSpec((1, 2, b), lambda g: (g, 0, 0)),
            pl.BlockSpec((4, b), lambda g: (0, g)),
            pl.BlockSpec((4, b), lambda g: (0, g)),
            pl.BlockSpec((1, 1, b), lambda g: (g, 0, 0)),
            pl.BlockSpec((1, 1, b), lambda g: (g, 0, 0)),
        ],
        out_specs=[
            pl.BlockSpec((1, 1), lambda g: (0, 0)),
            pl.BlockSpec((1, 1), lambda g: (0, 0)),
            pl.BlockSpec((1, 1), lambda g: (0, 0)),
        ],
        out_shape=[jax.ShapeDtypeStruct((1, 1), jnp.float32)] * 3,
        scratch_shapes=[pltpu.VMEM((1, 4), jnp.float32)],
    )(pred_cls, ctm, pb_t, bt_t, pi_row, ti_row)
    cls_s, reg_s, iou_s = out
    return (cls_s[0, 0], reg_s[0, 0], iou_s[0, 0])


# speedup vs baseline: 1.4996x; 1.1189x over previous
import functools

import jax
import jax.numpy as jnp
from jax.experimental import pallas as pl
from jax.experimental.pallas import tpu as pltpu

_ALPHA = 0.25
_B = 8192
_LOG2E = 1.4426950408889634
_LN2 = 0.6931471805599453


def _loss_kernel(num_blocks, num_classes,
                 x_ref, ctm_ref, lane_ref, pb_ref, bt_ref, pi_ref, ti_ref,
                 cls_ref, reg_ref, iou_ref, acc_ref):
    g = pl.program_id(0)

    ct = ctm_ref[0, 0:1, :]
    mval = ctm_ref[0, 1:2, :]
    fg = jnp.where((ct >= 0.0) & (ct != float(num_classes)), 1.0, 0.0)
    valid = jnp.where(ct >= 0.0, 1.0, 0.0) * (1.0 - mval)
    fg_part = jnp.sum(fg)

    ct_col = ct.T

    x = x_ref[...]
    lane = lane_ref[0:1, :]
    tau = lane == ct_col
    y = x * _LOG2E
    a = jnp.exp2(-jnp.abs(y))
    l2 = jnp.log2(1.0 + a)
    ry = jnp.maximum(y, 0.0)
    ce0 = ry + l2
    ty = jnp.where(tau, y, 0.0)
    ce2 = ce0 - ty
    arg = (ce0 - y) + ty
    fsq = jnp.exp2(-2.0 * arg)
    at = jnp.where(tau, _ALPHA, 1.0 - _ALPHA)
    focal2 = (at * ce2 * fsq).astype(jnp.bfloat16)
    vrow = valid.astype(jnp.bfloat16)
    colsum = jax.lax.dot_general(
        vrow, focal2, (((1,), (0,)), ((), ())),
        preferred_element_type=jnp.float32)
    cls_part = jnp.sum(colsum) * _LN2

    pb = pb_ref[...]
    bt = bt_ref[...]
    px0, py0, px1, py1 = pb[0:1], pb[1:2], pb[2:3], pb[3:4]
    tx0, ty0, tx1, ty1 = bt[0:1], bt[1:2], bt[2:3], bt[3:4]
    area1 = (px1 - px0) * (py1 - py0)
    area2 = (tx1 - tx0) * (ty1 - ty0)
    iw = jnp.maximum(jnp.minimum(px1, tx1) - jnp.maximum(px0, tx0), 0.0)
    ih = jnp.maximum(jnp.minimum(py1, ty1) - jnp.maximum(py0, ty0), 0.0)
    inter = iw * ih
    union = area1 + area2 - inter
    ew = jnp.maximum(px1, tx1) - jnp.minimum(px0, tx0)
    eh = jnp.maximum(py1, ty1) - jnp.minimum(py0, ty0)
    area_e = jnp.maximum(ew, 0.0) * jnp.maximum(eh, 0.0)
    giou = inter / union - (area_e - union) / area_e
    reg_part = jnp.sum((1.0 - giou) * fg)

    pi = pi_ref[0]
    ti = ti_ref[0]
    ai = jnp.exp2(jnp.abs(pi) * (-_LOG2E))
    bce = jnp.maximum(pi, 0.0) - pi * ti + jnp.log2(1.0 + ai) * _LN2
    iou_part = jnp.sum(bce * fg)

    part = jnp.concatenate(
        [jnp.full((1, 1), v, jnp.float32)
         for v in (cls_part, reg_part, iou_part, fg_part)], axis=1)

    @pl.when(g == 0)
    def _():
        acc_ref[...] = jnp.zeros((1, 4), jnp.float32)

    tot = acc_ref[...] + part
    acc_ref[...] = tot

    @pl.when(g == num_blocks - 1)
    def _():
        nf = jnp.maximum(tot[0:1, 3:4], 1.0)
        cls_ref[...] = tot[0:1, 0:1] / nf
        reg_ref[...] = tot[0:1, 1:2] / nf
        iou_ref[...] = tot[0:1, 2:3] / nf


def kernel(pred_cls, pred_box, pred_iou, cls_targets, box_targets,
           iou_targets, mask):
    n, c = pred_cls.shape
    b = _B
    nb = n // b
    ctm = jnp.stack([cls_targets.astype(jnp.float32),
                     mask.astype(jnp.float32)], axis=0)
    ctm = ctm.reshape(2, nb, b).transpose(1, 0, 2)
    pb_t = pred_box.T
    bt_t = box_targets.T
    pi_row = pred_iou.reshape(nb, 1, b)
    ti_row = iou_targets.reshape(nb, 1, b)
    lane_const = jnp.arange(c, dtype=jnp.float32).reshape(1, c) * jnp.ones(
        (8, 1), jnp.float32)

    out = pl.pallas_call(
        functools.partial(_loss_kernel, nb, c),
        grid=(nb,),
        in_specs=[
            pl.BlockSpec((b, c), lambda g: (g, 0)),
            pl.BlockSpec((1, 2, b), lambda g: (g, 0, 0)),
            pl.BlockSpec((8, c), lambda g: (0, 0)),
            pl.BlockSpec((4, b), lambda g: (0, g)),
            pl.BlockSpec((4, b), lambda g: (0, g)),
            pl.BlockSpec((1, 1, b), lambda g: (g, 0, 0)),
            pl.BlockSpec((1, 1, b), lambda g: (g, 0, 0)),
        ],
        out_specs=[
            pl.BlockSpec((1, 1), lambda g: (0, 0)),
            pl.BlockSpec((1, 1), lambda g: (0, 0)),
            pl.BlockSpec((1, 1), lambda g: (0, 0)),
        ],
        out_shape=[jax.ShapeDtypeStruct((1, 1), jnp.float32)] * 3,
        scratch_shapes=[pltpu.VMEM((1, 4), jnp.float32)],
    )(pred_cls, ctm, lane_const, pb_t, bt_t, pi_row, ti_row)
    cls_s, reg_s, iou_s = out
    return (cls_s[0, 0], reg_s[0, 0], iou_s[0, 0])


# speedup vs baseline: 1.6991x; 1.1330x over previous
import functools

import jax
import jax.numpy as jnp
from jax.experimental import pallas as pl
from jax.experimental.pallas import tpu as pltpu

_ALPHA = 0.25
_B = 8192
_LOG2E = 1.4426950408889634
_LN2 = 0.6931471805599453


def _loss_kernel(num_blocks, num_classes,
                 x_ref, ctm_ref, lane_ref, pb_ref, bt_ref, pi_ref, ti_ref,
                 cls_ref, reg_ref, iou_ref, acc_ref):
    g = pl.program_id(0)

    ct = ctm_ref[0, 0:1, :]
    mval = ctm_ref[0, 1:2, :]
    fg = jnp.where((ct >= 0.0) & (ct != float(num_classes)), 1.0, 0.0)
    valid = jnp.where(ct >= 0.0, 1.0, 0.0) * (1.0 - mval)
    fg_part = jnp.sum(fg)

    ct_col = ct.T

    x = x_ref[...]
    lane = lane_ref[0:1, :]
    tau = lane == ct_col.astype(jnp.bfloat16)
    y = x * _LOG2E
    a = jnp.exp2(-jnp.abs(y))
    l2b = jnp.log2(1.0 + a).astype(jnp.bfloat16)
    yb = y.astype(jnp.bfloat16)
    ryb = jnp.maximum(yb, jnp.bfloat16(0.0))
    ce0 = ryb + l2b
    nyb = ryb - yb
    u2 = nyb + l2b
    ty = jnp.where(tau, yb, jnp.bfloat16(0.0))
    ce2 = ce0 - ty
    arg = u2 + ty
    fsq = jnp.exp2(jnp.bfloat16(-2.0) * arg)
    at = jnp.where(tau, jnp.bfloat16(_ALPHA), jnp.bfloat16(1.0 - _ALPHA))
    focal2 = at * ce2 * fsq
    vrow = valid.astype(jnp.bfloat16)
    colsum = jax.lax.dot_general(
        vrow, focal2, (((1,), (0,)), ((), ())),
        preferred_element_type=jnp.float32)
    cls_part = jnp.sum(colsum) * _LN2

    pb = pb_ref[...]
    bt = bt_ref[...]
    px0, py0, px1, py1 = pb[0:1], pb[1:2], pb[2:3], pb[3:4]
    tx0, ty0, tx1, ty1 = bt[0:1], bt[1:2], bt[2:3], bt[3:4]
    area1 = (px1 - px0) * (py1 - py0)
    area2 = (tx1 - tx0) * (ty1 - ty0)
    iw = jnp.maximum(jnp.minimum(px1, tx1) - jnp.maximum(px0, tx0), 0.0)
    ih = jnp.maximum(jnp.minimum(py1, ty1) - jnp.maximum(py0, ty0), 0.0)
    inter = iw * ih
    union = area1 + area2 - inter
    ew = jnp.maximum(px1, tx1) - jnp.minimum(px0, tx0)
    eh = jnp.maximum(py1, ty1) - jnp.minimum(py0, ty0)
    area_e = jnp.maximum(ew, 0.0) * jnp.maximum(eh, 0.0)
    giou = inter / union - (area_e - union) / area_e
    reg_part = jnp.sum((1.0 - giou) * fg)

    pi = pi_ref[0]
    ti = ti_ref[0]
    ai = jnp.exp2(jnp.abs(pi) * (-_LOG2E))
    bce = jnp.maximum(pi, 0.0) - pi * ti + jnp.log2(1.0 + ai) * _LN2
    iou_part = jnp.sum(bce * fg)

    part = jnp.concatenate(
        [jnp.full((1, 1), v, jnp.float32)
         for v in (cls_part, reg_part, iou_part, fg_part)], axis=1)

    @pl.when(g == 0)
    def _():
        acc_ref[...] = jnp.zeros((1, 4), jnp.float32)

    tot = acc_ref[...] + part
    acc_ref[...] = tot

    @pl.when(g == num_blocks - 1)
    def _():
        nf = jnp.maximum(tot[0:1, 3:4], 1.0)
        cls_ref[...] = tot[0:1, 0:1] / nf
        reg_ref[...] = tot[0:1, 1:2] / nf
        iou_ref[...] = tot[0:1, 2:3] / nf


def kernel(pred_cls, pred_box, pred_iou, cls_targets, box_targets,
           iou_targets, mask):
    n, c = pred_cls.shape
    b = _B
    nb = n // b
    ctm = jnp.stack([cls_targets.astype(jnp.float32),
                     mask.astype(jnp.float32)], axis=0)
    ctm = ctm.reshape(2, nb, b).transpose(1, 0, 2)
    pb_t = pred_box.T
    bt_t = box_targets.T
    pi_row = pred_iou.reshape(nb, 1, b)
    ti_row = iou_targets.reshape(nb, 1, b)
    lane_const = jnp.arange(c, dtype=jnp.bfloat16).reshape(1, c) * jnp.ones(
        (8, 1), jnp.bfloat16)

    out = pl.pallas_call(
        functools.partial(_loss_kernel, nb, c),
        grid=(nb,),
        in_specs=[
            pl.BlockSpec((b, c), lambda g: (g, 0)),
            pl.BlockSpec((1, 2, b), lambda g: (g, 0, 0)),
            pl.BlockSpec((8, c), lambda g: (0, 0)),
            pl.BlockSpec((4, b), lambda g: (0, g)),
            pl.BlockSpec((4, b), lambda g: (0, g)),
            pl.BlockSpec((1, 1, b), lambda g: (g, 0, 0)),
            pl.BlockSpec((1, 1, b), lambda g: (g, 0, 0)),
        ],
        out_specs=[
            pl.BlockSpec((1, 1), lambda g: (0, 0)),
            pl.BlockSpec((1, 1), lambda g: (0, 0)),
            pl.BlockSpec((1, 1), lambda g: (0, 0)),
        ],
        out_shape=[jax.ShapeDtypeStruct((1, 1), jnp.float32)] * 3,
        scratch_shapes=[pltpu.VMEM((1, 4), jnp.float32)],
    )(pred_cls, ctm, lane_const, pb_t, bt_t, pi_row, ti_row)
    cls_s, reg_s, iou_s = out
    return (cls_s[0, 0], reg_s[0, 0], iou_s[0, 0])
